# baseline (speedup 1.0000x reference)
import functools

import jax
import jax.numpy as jnp
import numpy as np
from jax import lax
from jax.experimental import pallas as pl
from jax.experimental.pallas import tpu as pltpu
from jax.experimental.pallas import tpu_sc as plsc

N = 10000
N_PAD = 10240
E = 320000
F = 64
H = 64
D_IN = 128

NC = 2
NS = 16
NW = NC * NS
CHUNK = 128
CPW = 80
E_PAD = NW * CPW * CHUNK
ROWS_PT = N_PAD // NS

_mesh = plsc.VectorSubcoreMesh(core_axis_name="c", subcore_axis_name="s")


def _sc_degrees(src_c, dst_c):
    @functools.partial(
        pl.kernel,
        out_type=(jax.ShapeDtypeStruct((NC, N_PAD), jnp.float32),
                  jax.ShapeDtypeStruct((NC, N_PAD), jnp.float32)),
        mesh=_mesh,
        scratch_types=[
            pltpu.VMEM((CPW, CHUNK), jnp.int32),
            pltpu.VMEM((CPW, CHUNK), jnp.int32),
            pltpu.VMEM((CHUNK,), jnp.float32),
            pltpu.VMEM((ROWS_PT,), jnp.float32),
            pltpu.VMEM_SHARED((N_PAD,), jnp.float32),
            pltpu.VMEM_SHARED((N_PAD,), jnp.float32),
            pltpu.SemaphoreType.DMA,
            pltpu.SemaphoreType.DMA,
        ],
    )
    def k(src_hbm, dst_hbm, dsrc_hbm, ddst_hbm,
          src_v, dst_v, ones_v, stage_v, hsrc_sh, hdst_sh, sa, sb):
        c = lax.axis_index("c")
        s = lax.axis_index("s")
        w = c * NS + s
        base = s * ROWS_PT

        def fill_ones(i, carry):
            ones_v[pl.ds(i * 16, 16)] = jnp.ones((16,), jnp.float32)
            return carry
        lax.fori_loop(0, CHUNK // 16, fill_ones, 0)

        def fill_zero(i, carry):
            stage_v[pl.ds(i * 16, 16)] = jnp.zeros((16,), jnp.float32)
            return carry
        lax.fori_loop(0, ROWS_PT // 16, fill_zero, 0)

        pltpu.sync_copy(stage_v, hsrc_sh.at[pl.ds(base, ROWS_PT)])
        pltpu.sync_copy(stage_v, hdst_sh.at[pl.ds(base, ROWS_PT)])

        pltpu.sync_copy(src_hbm.at[w], src_v)
        pltpu.sync_copy(dst_hbm.at[w], dst_v)
        plsc.subcore_barrier()

        def body(j, carry):
            @pl.when(j >= 2)
            def _():
                pltpu.make_async_copy(ones_v, hsrc_sh.at[src_v.at[0]], sa).wait()
                pltpu.make_async_copy(ones_v, hdst_sh.at[dst_v.at[0]], sb).wait()
            pltpu.async_copy(ones_v, hsrc_sh.at[src_v.at[j]], sa, add=True)
            pltpu.async_copy(ones_v, hdst_sh.at[dst_v.at[j]], sb, add=True)
            return carry
        lax.fori_loop(0, CPW, body, 0)
        for _ in range(2):
            pltpu.make_async_copy(ones_v, hsrc_sh.at[src_v.at[0]], sa).wait()
            pltpu.make_async_copy(ones_v, hdst_sh.at[dst_v.at[0]], sb).wait()
        plsc.subcore_barrier()

        pltpu.sync_copy(hsrc_sh.at[pl.ds(base, ROWS_PT)], stage_v)
        pltpu.sync_copy(stage_v, dsrc_hbm.at[c, pl.ds(base, ROWS_PT)])
        pltpu.sync_copy(hdst_sh.at[pl.ds(base, ROWS_PT)], stage_v)
        pltpu.sync_copy(stage_v, ddst_hbm.at[c, pl.ds(base, ROWS_PT)])

    return k(src_c, dst_c)


def _sc_aggregate(t, src_c, dst_c, d, chunk, dtype):
    cpw = (CPW * CHUNK) // chunk
    vw = 32 if dtype == jnp.bfloat16 else 16
    nbuf = 4
    assert cpw % nbuf == 0

    @functools.partial(
        pl.kernel,
        out_type=jax.ShapeDtypeStruct((NC, N_PAD, d), dtype),
        mesh=_mesh,
        compiler_params=pltpu.CompilerParams(use_tc_tiling_on_sc=False),
        scratch_types=[
            pltpu.VMEM((cpw, chunk), jnp.int32),
            pltpu.VMEM((cpw, chunk), jnp.int32),
            pltpu.VMEM((nbuf, chunk, d), dtype),
            pltpu.VMEM_SHARED((N_PAD, d), dtype),
        ] + [pltpu.SemaphoreType.DMA] * (2 * nbuf),
    )
    def k(t_hbm, src_hbm, dst_hbm, out_hbm,
          src_v, dst_v, rows_v, agg_sh, *sems):
        gsem = sems[:nbuf]
        ssem = sems[nbuf:]
        c = lax.axis_index("c")
        s = lax.axis_index("s")
        w = c * NS + s
        base = s * ROWS_PT

        def zrow(i, carry):
            def zcol(j, carry2):
                rows_v[0, i, pl.ds(j * vw, vw)] = jnp.zeros((vw,), dtype)
                return carry2
            return lax.fori_loop(0, d // vw, zcol, carry)
        lax.fori_loop(0, chunk, zrow, 0)

        for i in range(ROWS_PT // chunk):
            pltpu.async_copy(rows_v.at[0],
                             agg_sh.at[pl.ds(base + i * chunk, chunk)],
                             gsem[0])
        pltpu.sync_copy(src_hbm.at[w], src_v)
        pltpu.sync_copy(dst_hbm.at[w], dst_v)
        for i in range(ROWS_PT // chunk):
            pltpu.make_async_copy(rows_v.at[0],
                                  agg_sh.at[pl.ds(base, chunk)],
                                  gsem[0]).wait()
        plsc.subcore_barrier()

        pltpu.async_copy(t_hbm.at[src_v.at[0]], rows_v.at[0], gsem[0])

        def outer(g, carry):
            for b in range(nbuf):
                j = nbuf * g + b
                nb = (b + 1) % nbuf
                pltpu.make_async_copy(t_hbm.at[src_v.at[0]],
                                      rows_v.at[b], gsem[b]).wait()
                pltpu.async_copy(rows_v.at[b], agg_sh.at[dst_v.at[j]],
                                 ssem[b], add=True)

                @pl.when(j + 1 < cpw)
                def _():
                    @pl.when(j >= nbuf - 1)
                    def _():
                        pltpu.make_async_copy(
                            rows_v.at[nb], agg_sh.at[dst_v.at[0]],
                            ssem[nb]).wait()
                    pltpu.async_copy(t_hbm.at[src_v.at[j + 1]],
                                     rows_v.at[nb], gsem[nb])
            return carry
        lax.fori_loop(0, cpw // nbuf, outer, 0)
        for i in range(nbuf):
            pltpu.make_async_copy(rows_v.at[(cpw - nbuf + i) % nbuf],
                                  agg_sh.at[dst_v.at[0]],
                                  ssem[(cpw - nbuf + i) % nbuf]).wait()
        plsc.subcore_barrier()

        for i in range(ROWS_PT // chunk):
            b = i % 2
            if i >= 2:
                pltpu.make_async_copy(
                    rows_v.at[b], out_hbm.at[c, pl.ds(base, chunk)],
                    gsem[b]).wait()
            pltpu.sync_copy(agg_sh.at[pl.ds(base + i * chunk, chunk)],
                            rows_v.at[b])
            pltpu.async_copy(rows_v.at[b],
                             out_hbm.at[c, pl.ds(base + i * chunk, chunk)],
                             gsem[b])
        for i in (ROWS_PT // chunk - 2, ROWS_PT // chunk - 1):
            pltpu.make_async_copy(rows_v.at[i % 2],
                                  out_hbm.at[c, pl.ds(base, chunk)],
                                  gsem[i % 2]).wait()

    return k(t, src_c, dst_c)


BLK = 1024
GRID = N_PAD // BLK
SUB = BLK // 128
NLANE = N_PAD // 128


def _norm_diag(dref, r):
    i = pl.program_id(0) * SUB + r
    deg = dref[0, pl.ds(i, 1), :] + dref[1, pl.ds(i, 1), :]
    nvec = lax.rsqrt(jnp.maximum(deg, 1.0))
    rr = lax.broadcasted_iota(jnp.int32, (128, 128), 0)
    cc = lax.broadcasted_iota(jnp.int32, (128, 128), 1)
    return jnp.where(rr == cc, nvec, 0.0)


def _tc_matmul0(xcat, W0):
    def body(x_ref, w_ref, u_ref):
        u_ref[...] = jnp.dot(x_ref[...], w_ref[...],
                             preferred_element_type=jnp.float32)
    return pl.pallas_call(
        body,
        grid=(GRID,),
        in_specs=[
            pl.BlockSpec((BLK, D_IN), lambda i: (i, 0)),
            pl.BlockSpec((D_IN, H), lambda i: (0, 0)),
        ],
        out_specs=pl.BlockSpec((BLK, H), lambda i: (i, 0)),
        out_shape=jax.ShapeDtypeStruct((N_PAD, H), jnp.float32),
    )(xcat, W0)


def _tc_scale0(dsq, u0):
    def body(ds_ref, u_ref, t_ref):
        for r in range(SUB):
            dns = _norm_diag(ds_ref, r)
            t_ref[pl.ds(r * 128, 128), :] = jnp.dot(
                dns, u_ref[pl.ds(r * 128, 128), :],
                preferred_element_type=jnp.float32).astype(jnp.bfloat16)
    return pl.pallas_call(
        body,
        grid=(GRID,),
        in_specs=[
            pl.BlockSpec((NC, NLANE, 128), lambda i: (0, 0, 0)),
            pl.BlockSpec((BLK, H), lambda i: (i, 0)),
        ],
        out_specs=pl.BlockSpec((BLK, H), lambda i: (i, 0)),
        out_shape=jax.ShapeDtypeStruct((N_PAD, H), jnp.bfloat16),
    )(dsq, u0)


def _tc_dense1(ddq, dsq, agg0, b0_2d, W1):
    def body(dd_ref, ds_ref, a0, a1, b_ref, w_ref, t_ref):
        agg = a0[0].astype(jnp.float32) + a1[0].astype(jnp.float32)
        for r in range(SUB):
            dnd = _norm_diag(dd_ref, r)
            dns = _norm_diag(ds_ref, r)
            a_r = agg[r * 128:(r + 1) * 128, :]
            out0 = jnp.maximum(
                jnp.dot(dnd, a_r, preferred_element_type=jnp.float32)
                + b_ref[...], 0.0)
            t_ref[pl.ds(r * 128, 128), :] = jnp.dot(
                jnp.dot(dns, out0, preferred_element_type=jnp.float32),
                w_ref[...], preferred_element_type=jnp.float32
            ).astype(jnp.bfloat16)
    return pl.pallas_call(
        body,
        grid=(GRID,),
        in_specs=[
            pl.BlockSpec((NC, NLANE, 128), lambda i: (0, 0, 0)),
            pl.BlockSpec((NC, NLANE, 128), lambda i: (0, 0, 0)),
            pl.BlockSpec((1, BLK, H), lambda i: (0, i, 0)),
            pl.BlockSpec((1, BLK, H), lambda i: (1, i, 0)),
            pl.BlockSpec((1, H), lambda i: (0, 0)),
            pl.BlockSpec((H, D_IN), lambda i: (0, 0)),
        ],
        out_specs=pl.BlockSpec((BLK, D_IN), lambda i: (i, 0)),
        out_shape=jax.ShapeDtypeStruct((N_PAD, D_IN), jnp.bfloat16),
    )(ddq, dsq, agg0, agg0, b0_2d, W1)


def _tc_final(ddq, agg1, b1_2d, xcat, sel):
    def body(dd_ref, a0, a1, b_ref, x_ref, s_ref, o_ref):
        agg = a0[0].astype(jnp.float32) + a1[0].astype(jnp.float32)
        for r in range(SUB):
            dnd = _norm_diag(dd_ref, r)
            a_r = agg[r * 128:(r + 1) * 128, :]
            h = jnp.dot(dnd, a_r,
                        preferred_element_type=jnp.float32) + b_ref[...]
            o_ref[pl.ds(r * 128, 128), :] = jnp.dot(
                h * x_ref[r * 128:(r + 1) * 128, :], s_ref[...],
                preferred_element_type=jnp.float32)
    return pl.pallas_call(
        body,
        grid=(GRID,),
        in_specs=[
            pl.BlockSpec((NC, NLANE, 128), lambda i: (0, 0, 0)),
            pl.BlockSpec((1, BLK, D_IN), lambda i: (0, i, 0)),
            pl.BlockSpec((1, BLK, D_IN), lambda i: (1, i, 0)),
            pl.BlockSpec((1, D_IN), lambda i: (0, 0)),
            pl.BlockSpec((BLK, D_IN), lambda i: (i, 0)),
            pl.BlockSpec((D_IN, F), lambda i: (0, 0)),
        ],
        out_specs=pl.BlockSpec((BLK, F), lambda i: (i, 0)),
        out_shape=jax.ShapeDtypeStruct((N_PAD, F), jnp.float32),
    )(ddq, agg1, agg1, b1_2d, xcat, sel)


def kernel(x_u, x_s, edge_index, W0, b0, W1, b1):
    ei = edge_index.astype(jnp.int32)
    pad_idx = jnp.asarray(
        N + (np.arange(E_PAD - E, dtype=np.int32) % (N_PAD - N)))
    src = jnp.concatenate([ei[0], pad_idx])
    dst = jnp.concatenate([ei[1], pad_idx])
    src_c = src.reshape(NW, CPW, CHUNK)
    dst_c = dst.reshape(NW, CPW, CHUNK)

    xcat = jnp.pad(jnp.concatenate([x_u, x_s], axis=1),
                   ((0, N_PAD - N), (0, 0)))
    b0_2d = b0.reshape(1, H)
    b1_2d = b1.reshape(1, D_IN)
    eye = jnp.eye(F, dtype=jnp.float32)
    sel = jnp.concatenate([eye, eye], axis=0)

    u0 = _tc_matmul0(xcat, W0)
    dsrc_p, ddst_p = _sc_degrees(src_c, dst_c)
    dsq = dsrc_p.reshape(NC, NLANE, 128)
    ddq = ddst_p.reshape(NC, NLANE, 128)

    t0 = _tc_scale0(dsq, u0)
    agg0 = _sc_aggregate(t0, src_c, dst_c, H, CHUNK, jnp.bfloat16)
    t1 = _tc_dense1(ddq, dsq, agg0, b0_2d, W1)
    agg1 = _sc_aggregate(t1, src_c, dst_c, D_IN, CHUNK, jnp.bfloat16)
    pred = _tc_final(ddq, agg1, b1_2d, xcat, sel)
    return pred[:N]


# speedup vs baseline: 1.0017x; 1.0017x over previous
import functools

import jax
import jax.numpy as jnp
import numpy as np
from jax import lax
from jax.experimental import pallas as pl
from jax.experimental.pallas import tpu as pltpu
from jax.experimental.pallas import tpu_sc as plsc

N = 10000
N_PAD = 10240
E = 320000
F = 64
H = 64
D_IN = 128

NC = 2
NS = 16
NW = NC * NS
CHUNK = 128
CPW = 80
E_PAD = NW * CPW * CHUNK
ROWS_PT = N_PAD // NS

_mesh = plsc.VectorSubcoreMesh(core_axis_name="c", subcore_axis_name="s")


def _sc_degrees(src_c, dst_c):
    @functools.partial(
        pl.kernel,
        out_type=(jax.ShapeDtypeStruct((NC, N_PAD), jnp.float32),
                  jax.ShapeDtypeStruct((NC, N_PAD), jnp.float32)),
        mesh=_mesh,
        scratch_types=[
            pltpu.VMEM((CPW, CHUNK), jnp.int32),
            pltpu.VMEM((CPW, CHUNK), jnp.int32),
            pltpu.VMEM((CHUNK,), jnp.float32),
            pltpu.VMEM((ROWS_PT,), jnp.float32),
            pltpu.VMEM_SHARED((N_PAD,), jnp.float32),
            pltpu.VMEM_SHARED((N_PAD,), jnp.float32),
            pltpu.SemaphoreType.DMA,
            pltpu.SemaphoreType.DMA,
        ],
    )
    def k(src_hbm, dst_hbm, dsrc_hbm, ddst_hbm,
          src_v, dst_v, ones_v, stage_v, hsrc_sh, hdst_sh, sa, sb):
        c = lax.axis_index("c")
        s = lax.axis_index("s")
        w = c * NS + s
        base = s * ROWS_PT

        def fill_ones(i, carry):
            ones_v[pl.ds(i * 16, 16)] = jnp.ones((16,), jnp.float32)
            return carry
        lax.fori_loop(0, CHUNK // 16, fill_ones, 0)

        def fill_zero(i, carry):
            stage_v[pl.ds(i * 16, 16)] = jnp.zeros((16,), jnp.float32)
            return carry
        lax.fori_loop(0, ROWS_PT // 16, fill_zero, 0)

        pltpu.sync_copy(stage_v, hsrc_sh.at[pl.ds(base, ROWS_PT)])
        pltpu.sync_copy(stage_v, hdst_sh.at[pl.ds(base, ROWS_PT)])

        pltpu.sync_copy(src_hbm.at[w], src_v)
        pltpu.sync_copy(dst_hbm.at[w], dst_v)
        plsc.subcore_barrier()

        def body(j, carry):
            @pl.when(j >= 2)
            def _():
                pltpu.make_async_copy(ones_v, hsrc_sh.at[src_v.at[0]], sa).wait()
                pltpu.make_async_copy(ones_v, hdst_sh.at[dst_v.at[0]], sb).wait()
            pltpu.async_copy(ones_v, hsrc_sh.at[src_v.at[j]], sa, add=True)
            pltpu.async_copy(ones_v, hdst_sh.at[dst_v.at[j]], sb, add=True)
            return carry
        lax.fori_loop(0, CPW, body, 0)
        for _ in range(2):
            pltpu.make_async_copy(ones_v, hsrc_sh.at[src_v.at[0]], sa).wait()
            pltpu.make_async_copy(ones_v, hdst_sh.at[dst_v.at[0]], sb).wait()
        plsc.subcore_barrier()

        pltpu.sync_copy(hsrc_sh.at[pl.ds(base, ROWS_PT)], stage_v)
        pltpu.sync_copy(stage_v, dsrc_hbm.at[c, pl.ds(base, ROWS_PT)])
        pltpu.sync_copy(hdst_sh.at[pl.ds(base, ROWS_PT)], stage_v)
        pltpu.sync_copy(stage_v, ddst_hbm.at[c, pl.ds(base, ROWS_PT)])

    return k(src_c, dst_c)


def _sc_aggregate(t, src_c, dst_c, d, chunk, dtype):
    cpw = (CPW * CHUNK) // chunk
    vw = 32 if dtype == jnp.bfloat16 else 16
    nbuf = 4
    assert cpw % nbuf == 0

    @functools.partial(
        pl.kernel,
        out_type=jax.ShapeDtypeStruct((NC, N_PAD, d), dtype),
        mesh=_mesh,
        compiler_params=pltpu.CompilerParams(use_tc_tiling_on_sc=False),
        scratch_types=[
            pltpu.VMEM((cpw, chunk), jnp.int32),
            pltpu.VMEM((cpw, chunk), jnp.int32),
            pltpu.VMEM((nbuf, chunk, d), dtype),
            pltpu.VMEM_SHARED((N_PAD, d), dtype),
        ] + [pltpu.SemaphoreType.DMA] * (2 * nbuf),
    )
    def k(t_hbm, src_hbm, dst_hbm, out_hbm,
          src_v, dst_v, rows_v, agg_sh, *sems):
        gsem = sems[:nbuf]
        ssem = sems[nbuf:]
        c = lax.axis_index("c")
        s = lax.axis_index("s")
        w = c * NS + s
        base = s * ROWS_PT

        def zrow(i, carry):
            def zcol(j, carry2):
                rows_v[0, i, pl.ds(j * vw, vw)] = jnp.zeros((vw,), dtype)
                return carry2
            return lax.fori_loop(0, d // vw, zcol, carry)
        lax.fori_loop(0, chunk, zrow, 0)

        for i in range(ROWS_PT // chunk):
            pltpu.async_copy(rows_v.at[0],
                             agg_sh.at[pl.ds(base + i * chunk, chunk)],
                             gsem[0])
        pltpu.sync_copy(src_hbm.at[w], src_v)
        pltpu.sync_copy(dst_hbm.at[w], dst_v)
        for i in range(ROWS_PT // chunk):
            pltpu.make_async_copy(rows_v.at[0],
                                  agg_sh.at[pl.ds(base, chunk)],
                                  gsem[0]).wait()
        plsc.subcore_barrier()

        pltpu.async_copy(t_hbm.at[src_v.at[0]], rows_v.at[0], gsem[0])

        def outer(g, carry):
            for b in range(nbuf):
                j = nbuf * g + b
                nb = (b + 1) % nbuf
                pltpu.make_async_copy(t_hbm.at[src_v.at[0]],
                                      rows_v.at[b], gsem[b]).wait()
                pltpu.async_copy(rows_v.at[b], agg_sh.at[dst_v.at[j]],
                                 ssem[b], add=True)

                @pl.when(j + 1 < cpw)
                def _():
                    @pl.when(j >= nbuf - 1)
                    def _():
                        pltpu.make_async_copy(
                            rows_v.at[nb], agg_sh.at[dst_v.at[0]],
                            ssem[nb]).wait()
                    pltpu.async_copy(t_hbm.at[src_v.at[j + 1]],
                                     rows_v.at[nb], gsem[nb])
            return carry
        lax.fori_loop(0, cpw // nbuf, outer, 0)
        for i in range(nbuf):
            pltpu.make_async_copy(rows_v.at[(cpw - nbuf + i) % nbuf],
                                  agg_sh.at[dst_v.at[0]],
                                  ssem[(cpw - nbuf + i) % nbuf]).wait()
        plsc.subcore_barrier()

        for i in range(ROWS_PT // chunk):
            b = i % 2
            if i >= 2:
                pltpu.make_async_copy(
                    rows_v.at[b], out_hbm.at[c, pl.ds(base, chunk)],
                    gsem[b]).wait()
            pltpu.sync_copy(agg_sh.at[pl.ds(base + i * chunk, chunk)],
                            rows_v.at[b])
            pltpu.async_copy(rows_v.at[b],
                             out_hbm.at[c, pl.ds(base + i * chunk, chunk)],
                             gsem[b])
        for i in (ROWS_PT // chunk - 2, ROWS_PT // chunk - 1):
            pltpu.make_async_copy(rows_v.at[i % 2],
                                  out_hbm.at[c, pl.ds(base, chunk)],
                                  gsem[i % 2]).wait()

    return k(t, src_c, dst_c)


BLK = 1024
GRID = N_PAD // BLK
SUB = BLK // 128
NLANE = N_PAD // 128


def _norm_diag(dref, r):
    i = pl.program_id(0) * SUB + r
    deg = dref[0, pl.ds(i, 1), :] + dref[1, pl.ds(i, 1), :]
    nvec = lax.rsqrt(jnp.maximum(deg, 1.0))
    rr = lax.broadcasted_iota(jnp.int32, (128, 128), 0)
    cc = lax.broadcasted_iota(jnp.int32, (128, 128), 1)
    return jnp.where(rr == cc, nvec, 0.0)


def _tc_matmul0(xcat, W0):
    def body(x_ref, w_ref, u_ref):
        u_ref[...] = jnp.dot(x_ref[...], w_ref[...],
                             preferred_element_type=jnp.float32)
    return pl.pallas_call(
        body,
        grid=(GRID,),
        in_specs=[
            pl.BlockSpec((BLK, D_IN), lambda i: (i, 0)),
            pl.BlockSpec((D_IN, H), lambda i: (0, 0)),
        ],
        out_specs=pl.BlockSpec((BLK, H), lambda i: (i, 0)),
        out_shape=jax.ShapeDtypeStruct((N_PAD, H), jnp.float32),
    )(xcat, W0)


def _tc_scale0(dsq, u0):
    def body(ds_ref, u_ref, t_ref):
        u = u_ref[...]
        for r in range(SUB):
            dns = _norm_diag(ds_ref, r)
            t_ref[pl.ds(r * 128, 128), :] = jnp.dot(
                dns, u[r * 128:(r + 1) * 128, :],
                preferred_element_type=jnp.float32).astype(jnp.bfloat16)
    return pl.pallas_call(
        body,
        grid=(GRID,),
        in_specs=[
            pl.BlockSpec((NC, NLANE, 128), lambda i: (0, 0, 0)),
            pl.BlockSpec((BLK, H), lambda i: (i, 0)),
        ],
        out_specs=pl.BlockSpec((BLK, H), lambda i: (i, 0)),
        out_shape=jax.ShapeDtypeStruct((N_PAD, H), jnp.bfloat16),
    )(dsq, u0)


def _tc_dense1(ddq, dsq, agg0, b0_2d, W1):
    def body(dd_ref, ds_ref, a0, a1, b_ref, w_ref, t_ref):
        agg = a0[0].astype(jnp.float32) + a1[0].astype(jnp.float32)
        for r in range(SUB):
            dnd = _norm_diag(dd_ref, r)
            dns = _norm_diag(ds_ref, r)
            a_r = agg[r * 128:(r + 1) * 128, :]
            out0 = jnp.maximum(
                jnp.dot(dnd, a_r, preferred_element_type=jnp.float32)
                + b_ref[...], 0.0)
            t_ref[pl.ds(r * 128, 128), :] = jnp.dot(
                jnp.dot(dns, out0, preferred_element_type=jnp.float32),
                w_ref[...], preferred_element_type=jnp.float32
            ).astype(jnp.bfloat16)
    return pl.pallas_call(
        body,
        grid=(GRID,),
        in_specs=[
            pl.BlockSpec((NC, NLANE, 128), lambda i: (0, 0, 0)),
            pl.BlockSpec((NC, NLANE, 128), lambda i: (0, 0, 0)),
            pl.BlockSpec((1, BLK, H), lambda i: (0, i, 0)),
            pl.BlockSpec((1, BLK, H), lambda i: (1, i, 0)),
            pl.BlockSpec((1, H), lambda i: (0, 0)),
            pl.BlockSpec((H, D_IN), lambda i: (0, 0)),
        ],
        out_specs=pl.BlockSpec((BLK, D_IN), lambda i: (i, 0)),
        out_shape=jax.ShapeDtypeStruct((N_PAD, D_IN), jnp.bfloat16),
    )(ddq, dsq, agg0, agg0, b0_2d, W1)


def _tc_final(ddq, agg1, b1_2d, xcat, sel):
    def body(dd_ref, a0, a1, b_ref, x_ref, s_ref, o_ref):
        agg = a0[0].astype(jnp.float32) + a1[0].astype(jnp.float32)
        for r in range(SUB):
            dnd = _norm_diag(dd_ref, r)
            a_r = agg[r * 128:(r + 1) * 128, :]
            h = jnp.dot(dnd, a_r,
                        preferred_element_type=jnp.float32) + b_ref[...]
            o_ref[pl.ds(r * 128, 128), :] = jnp.dot(
                h * x_ref[r * 128:(r + 1) * 128, :], s_ref[...],
                preferred_element_type=jnp.float32)
    return pl.pallas_call(
        body,
        grid=(GRID,),
        in_specs=[
            pl.BlockSpec((NC, NLANE, 128), lambda i: (0, 0, 0)),
            pl.BlockSpec((1, BLK, D_IN), lambda i: (0, i, 0)),
            pl.BlockSpec((1, BLK, D_IN), lambda i: (1, i, 0)),
            pl.BlockSpec((1, D_IN), lambda i: (0, 0)),
            pl.BlockSpec((BLK, D_IN), lambda i: (i, 0)),
            pl.BlockSpec((D_IN, F), lambda i: (0, 0)),
        ],
        out_specs=pl.BlockSpec((BLK, F), lambda i: (i, 0)),
        out_shape=jax.ShapeDtypeStruct((N_PAD, F), jnp.float32),
    )(ddq, agg1, agg1, b1_2d, xcat, sel)


def kernel(x_u, x_s, edge_index, W0, b0, W1, b1):
    ei = edge_index.astype(jnp.int32)
    pad_idx = jnp.asarray(
        N + (np.arange(E_PAD - E, dtype=np.int32) % (N_PAD - N)))
    src = jnp.concatenate([ei[0], pad_idx])
    dst = jnp.concatenate([ei[1], pad_idx])
    src_c = src.reshape(NW, CPW, CHUNK)
    dst_c = dst.reshape(NW, CPW, CHUNK)

    xcat = jnp.pad(jnp.concatenate([x_u, x_s], axis=1),
                   ((0, N_PAD - N), (0, 0)))
    b0_2d = b0.reshape(1, H)
    b1_2d = b1.reshape(1, D_IN)
    eye = jnp.eye(F, dtype=jnp.float32)
    sel = jnp.concatenate([eye, eye], axis=0)

    u0 = _tc_matmul0(xcat, W0)
    dsrc_p, ddst_p = _sc_degrees(src_c, dst_c)
    dsq = dsrc_p.reshape(NC, NLANE, 128)
    ddq = ddst_p.reshape(NC, NLANE, 128)

    t0 = _tc_scale0(dsq, u0)
    agg0 = _sc_aggregate(t0, src_c, dst_c, H, CHUNK, jnp.bfloat16)
    t1 = _tc_dense1(ddq, dsq, agg0, b0_2d, W1)
    agg1 = _sc_aggregate(t1, src_c, dst_c, D_IN, CHUNK, jnp.bfloat16)
    pred = _tc_final(ddq, agg1, b1_2d, xcat, sel)
    return pred[:N]


# speedup vs baseline: 1.0317x; 1.0299x over previous
import functools

import jax
import jax.numpy as jnp
import numpy as np
from jax import lax
from jax.experimental import pallas as pl
from jax.experimental.pallas import tpu as pltpu
from jax.experimental.pallas import tpu_sc as plsc

N = 10000
N_PAD = 10240
E = 320000
F = 64
H = 64
D_IN = 128

NC = 2
NS = 16
NW = NC * NS
CHUNK = 128
CPW = 80
E_PAD = NW * CPW * CHUNK
ROWS_PT = N_PAD // NS

_mesh = plsc.VectorSubcoreMesh(core_axis_name="c", subcore_axis_name="s")


def _sc_degrees(src_c, dst_c):
    @functools.partial(
        pl.kernel,
        out_type=(jax.ShapeDtypeStruct((NC, N_PAD), jnp.float32),
                  jax.ShapeDtypeStruct((NC, N_PAD), jnp.float32)),
        mesh=_mesh,
        scratch_types=[
            pltpu.VMEM((CPW, CHUNK), jnp.int32),
            pltpu.VMEM((CPW, CHUNK), jnp.int32),
            pltpu.VMEM((CHUNK,), jnp.float32),
            pltpu.VMEM((ROWS_PT,), jnp.float32),
            pltpu.VMEM_SHARED((N_PAD,), jnp.float32),
            pltpu.VMEM_SHARED((N_PAD,), jnp.float32),
            pltpu.SemaphoreType.DMA,
            pltpu.SemaphoreType.DMA,
        ],
    )
    def k(src_hbm, dst_hbm, dsrc_hbm, ddst_hbm,
          src_v, dst_v, ones_v, stage_v, hsrc_sh, hdst_sh, sa, sb):
        c = lax.axis_index("c")
        s = lax.axis_index("s")
        w = c * NS + s
        base = s * ROWS_PT

        def fill_ones(i, carry):
            ones_v[pl.ds(i * 16, 16)] = jnp.ones((16,), jnp.float32)
            return carry
        lax.fori_loop(0, CHUNK // 16, fill_ones, 0)

        def fill_zero(i, carry):
            stage_v[pl.ds(i * 16, 16)] = jnp.zeros((16,), jnp.float32)
            return carry
        lax.fori_loop(0, ROWS_PT // 16, fill_zero, 0)

        pltpu.sync_copy(stage_v, hsrc_sh.at[pl.ds(base, ROWS_PT)])
        pltpu.sync_copy(stage_v, hdst_sh.at[pl.ds(base, ROWS_PT)])

        pltpu.sync_copy(src_hbm.at[w], src_v)
        pltpu.sync_copy(dst_hbm.at[w], dst_v)
        plsc.subcore_barrier()

        def body(j, carry):
            @pl.when(j >= 2)
            def _():
                pltpu.make_async_copy(ones_v, hsrc_sh.at[src_v.at[0]], sa).wait()
                pltpu.make_async_copy(ones_v, hdst_sh.at[dst_v.at[0]], sb).wait()
            pltpu.async_copy(ones_v, hsrc_sh.at[src_v.at[j]], sa, add=True)
            pltpu.async_copy(ones_v, hdst_sh.at[dst_v.at[j]], sb, add=True)
            return carry
        lax.fori_loop(0, CPW, body, 0)
        for _ in range(2):
            pltpu.make_async_copy(ones_v, hsrc_sh.at[src_v.at[0]], sa).wait()
            pltpu.make_async_copy(ones_v, hdst_sh.at[dst_v.at[0]], sb).wait()
        plsc.subcore_barrier()

        pltpu.sync_copy(hsrc_sh.at[pl.ds(base, ROWS_PT)], stage_v)
        pltpu.sync_copy(stage_v, dsrc_hbm.at[c, pl.ds(base, ROWS_PT)])
        pltpu.sync_copy(hdst_sh.at[pl.ds(base, ROWS_PT)], stage_v)
        pltpu.sync_copy(stage_v, ddst_hbm.at[c, pl.ds(base, ROWS_PT)])

    return k(src_c, dst_c)


def _sc_aggregate(t, src_c, dst_c, d, chunk, dtype):
    cpw = (CPW * CHUNK) // chunk
    vw = 32 if dtype == jnp.bfloat16 else 16
    nbuf = 4
    assert cpw % nbuf == 0

    @functools.partial(
        pl.kernel,
        out_type=jax.ShapeDtypeStruct((NC, N_PAD, d), dtype),
        mesh=_mesh,
        compiler_params=pltpu.CompilerParams(use_tc_tiling_on_sc=False),
        scratch_types=[
            pltpu.VMEM((cpw, chunk), jnp.int32),
            pltpu.VMEM((cpw, chunk), jnp.int32),
            pltpu.VMEM((nbuf, chunk, d), dtype),
            pltpu.VMEM_SHARED((N_PAD, d), dtype),
        ] + [pltpu.SemaphoreType.DMA] * (2 * nbuf),
    )
    def k(t_hbm, src_hbm, dst_hbm, out_hbm,
          src_v, dst_v, rows_v, agg_sh, *sems):
        gsem = sems[:nbuf]
        ssem = sems[nbuf:]
        c = lax.axis_index("c")
        s = lax.axis_index("s")
        w = c * NS + s
        base = s * ROWS_PT

        def zrow(i, carry):
            def zcol(j, carry2):
                rows_v[0, i, pl.ds(j * vw, vw)] = jnp.zeros((vw,), dtype)
                return carry2
            return lax.fori_loop(0, d // vw, zcol, carry)
        lax.fori_loop(0, chunk, zrow, 0)

        for i in range(ROWS_PT // chunk):
            pltpu.async_copy(rows_v.at[0],
                             agg_sh.at[pl.ds(base + i * chunk, chunk)],
                             gsem[0])
        pltpu.sync_copy(src_hbm.at[w], src_v)
        pltpu.sync_copy(dst_hbm.at[w], dst_v)
        for i in range(ROWS_PT // chunk):
            pltpu.make_async_copy(rows_v.at[0],
                                  agg_sh.at[pl.ds(base, chunk)],
                                  gsem[0]).wait()
        plsc.subcore_barrier()

        pltpu.async_copy(t_hbm.at[src_v.at[0]], rows_v.at[0], gsem[0])

        def outer(g, carry):
            for b in range(nbuf):
                j = nbuf * g + b
                nb = (b + 1) % nbuf
                pltpu.make_async_copy(t_hbm.at[src_v.at[0]],
                                      rows_v.at[b], gsem[b]).wait()
                pltpu.async_copy(rows_v.at[b], agg_sh.at[dst_v.at[j]],
                                 ssem[b], add=True)

                @pl.when(j + 1 < cpw)
                def _():
                    @pl.when(j >= nbuf - 1)
                    def _():
                        pltpu.make_async_copy(
                            rows_v.at[nb], agg_sh.at[dst_v.at[0]],
                            ssem[nb]).wait()
                    pltpu.async_copy(t_hbm.at[src_v.at[j + 1]],
                                     rows_v.at[nb], gsem[nb])
            return carry
        lax.fori_loop(0, cpw // nbuf, outer, 0)
        for i in range(nbuf):
            pltpu.make_async_copy(rows_v.at[(cpw - nbuf + i) % nbuf],
                                  agg_sh.at[dst_v.at[0]],
                                  ssem[(cpw - nbuf + i) % nbuf]).wait()
        plsc.subcore_barrier()

        for i in range(ROWS_PT // chunk):
            b = i % 2
            if i >= 2:
                pltpu.make_async_copy(
                    rows_v.at[b], out_hbm.at[c, pl.ds(base, chunk)],
                    gsem[b]).wait()
            pltpu.sync_copy(agg_sh.at[pl.ds(base + i * chunk, chunk)],
                            rows_v.at[b])
            pltpu.async_copy(rows_v.at[b],
                             out_hbm.at[c, pl.ds(base + i * chunk, chunk)],
                             gsem[b])
        for i in (ROWS_PT // chunk - 2, ROWS_PT // chunk - 1):
            pltpu.make_async_copy(rows_v.at[i % 2],
                                  out_hbm.at[c, pl.ds(base, chunk)],
                                  gsem[i % 2]).wait()

    return k(t, src_c, dst_c)


BLK = 1024
GRID = N_PAD // BLK
SUB = BLK // 128
NLANE = N_PAD // 128


def _norm_diag(dref, r):
    i = pl.program_id(0) * SUB + r
    deg = dref[0, pl.ds(i, 1), :] + dref[1, pl.ds(i, 1), :]
    nvec = lax.rsqrt(jnp.maximum(deg, 1.0))
    rr = lax.broadcasted_iota(jnp.int32, (128, 128), 0)
    cc = lax.broadcasted_iota(jnp.int32, (128, 128), 1)
    return jnp.where(rr == cc, nvec, 0.0)


def _tc_dense0(dsq, xcat, W0):
    def body(ds_ref, x_ref, w_ref, t_ref):
        u = jnp.dot(x_ref[...], w_ref[...],
                    preferred_element_type=jnp.float32)
        for r in range(SUB):
            dns = _norm_diag(ds_ref, r)
            t_ref[pl.ds(r * 128, 128), :] = jnp.dot(
                dns, u[r * 128:(r + 1) * 128, :],
                preferred_element_type=jnp.float32).astype(jnp.bfloat16)
    return pl.pallas_call(
        body,
        grid=(GRID,),
        in_specs=[
            pl.BlockSpec((NC, NLANE, 128), lambda i: (0, 0, 0)),
            pl.BlockSpec((BLK, D_IN), lambda i: (i, 0)),
            pl.BlockSpec((D_IN, H), lambda i: (0, 0)),
        ],
        out_specs=pl.BlockSpec((BLK, H), lambda i: (i, 0)),
        out_shape=jax.ShapeDtypeStruct((N_PAD, H), jnp.bfloat16),
    )(dsq, xcat, W0)


def _tc_dense1(ddq, dsq, agg0, b0_2d, W1):
    def body(dd_ref, ds_ref, a0, a1, b_ref, w_ref, t_ref):
        agg = a0[0].astype(jnp.float32) + a1[0].astype(jnp.float32)
        for r in range(SUB):
            dnd = _norm_diag(dd_ref, r)
            dns = _norm_diag(ds_ref, r)
            a_r = agg[r * 128:(r + 1) * 128, :]
            out0 = jnp.maximum(
                jnp.dot(dnd, a_r, preferred_element_type=jnp.float32)
                + b_ref[...], 0.0)
            t_ref[pl.ds(r * 128, 128), :] = jnp.dot(
                jnp.dot(dns, out0, preferred_element_type=jnp.float32),
                w_ref[...], preferred_element_type=jnp.float32
            ).astype(jnp.bfloat16)
    return pl.pallas_call(
        body,
        grid=(GRID,),
        in_specs=[
            pl.BlockSpec((NC, NLANE, 128), lambda i: (0, 0, 0)),
            pl.BlockSpec((NC, NLANE, 128), lambda i: (0, 0, 0)),
            pl.BlockSpec((1, BLK, H), lambda i: (0, i, 0)),
            pl.BlockSpec((1, BLK, H), lambda i: (1, i, 0)),
            pl.BlockSpec((1, H), lambda i: (0, 0)),
            pl.BlockSpec((H, D_IN), lambda i: (0, 0)),
        ],
        out_specs=pl.BlockSpec((BLK, D_IN), lambda i: (i, 0)),
        out_shape=jax.ShapeDtypeStruct((N_PAD, D_IN), jnp.bfloat16),
    )(ddq, dsq, agg0, agg0, b0_2d, W1)


def _tc_final(ddq, agg1, b1_2d, xcat, sel):
    def body(dd_ref, a0, a1, b_ref, x_ref, s_ref, o_ref):
        agg = a0[0].astype(jnp.float32) + a1[0].astype(jnp.float32)
        for r in range(SUB):
            dnd = _norm_diag(dd_ref, r)
            a_r = agg[r * 128:(r + 1) * 128, :]
            h = jnp.dot(dnd, a_r,
                        preferred_element_type=jnp.float32) + b_ref[...]
            o_ref[pl.ds(r * 128, 128), :] = jnp.dot(
                h * x_ref[r * 128:(r + 1) * 128, :], s_ref[...],
                preferred_element_type=jnp.float32)
    return pl.pallas_call(
        body,
        grid=(GRID,),
        in_specs=[
            pl.BlockSpec((NC, NLANE, 128), lambda i: (0, 0, 0)),
            pl.BlockSpec((1, BLK, D_IN), lambda i: (0, i, 0)),
            pl.BlockSpec((1, BLK, D_IN), lambda i: (1, i, 0)),
            pl.BlockSpec((1, D_IN), lambda i: (0, 0)),
            pl.BlockSpec((BLK, D_IN), lambda i: (i, 0)),
            pl.BlockSpec((D_IN, F), lambda i: (0, 0)),
        ],
        out_specs=pl.BlockSpec((BLK, F), lambda i: (i, 0)),
        out_shape=jax.ShapeDtypeStruct((N_PAD, F), jnp.float32),
    )(ddq, agg1, agg1, b1_2d, xcat, sel)


def kernel(x_u, x_s, edge_index, W0, b0, W1, b1):
    ei = edge_index.astype(jnp.int32)
    pad_idx = jnp.asarray(
        N + (np.arange(E_PAD - E, dtype=np.int32) % (N_PAD - N)))
    src = jnp.concatenate([ei[0], pad_idx])
    dst = jnp.concatenate([ei[1], pad_idx])
    src_c = src.reshape(NW, CPW, CHUNK)
    dst_c = dst.reshape(NW, CPW, CHUNK)

    xcat = jnp.pad(jnp.concatenate([x_u, x_s], axis=1),
                   ((0, N_PAD - N), (0, 0)))
    b0_2d = b0.reshape(1, H)
    b1_2d = b1.reshape(1, D_IN)
    eye = jnp.eye(F, dtype=jnp.float32)
    sel = jnp.concatenate([eye, eye], axis=0)

    dsrc_p, ddst_p = _sc_degrees(src_c, dst_c)
    dsq = dsrc_p.reshape(NC, NLANE, 128)
    ddq = ddst_p.reshape(NC, NLANE, 128)

    t0 = _tc_dense0(dsq, xcat, W0)
    agg0 = _sc_aggregate(t0, src_c, dst_c, H, CHUNK, jnp.bfloat16)
    t1 = _tc_dense1(ddq, dsq, agg0, b0_2d, W1)
    agg1 = _sc_aggregate(t1, src_c, dst_c, D_IN, CHUNK, jnp.bfloat16)
    pred = _tc_final(ddq, agg1, b1_2d, xcat, sel)
    return pred[:N]
